# baseline (speedup 1.0000x reference)
import functools

import jax
import jax.numpy as jnp
from jax import lax
from jax.experimental import pallas as pl
from jax.experimental.pallas import tpu as pltpu
from jax.experimental.pallas import tpu_sc as plsc

_BIN = 128


def _elu(x):
    return jnp.where(x > 0, x, jnp.exp(x) - 1.0)


def _pos_body(bi_ref, pos_ref, ordt_ref, *, nb: int):
    b = pl.program_id(0)
    n = bi_ref.shape[-1]
    bi = bi_ref[0]
    rows = lax.broadcasted_iota(jnp.int32, (nb, n), 0)
    oht = (bi == rows).astype(jnp.float32)
    x = oht
    k = 1
    while k < n:
        x = x + jnp.concatenate(
            [jnp.zeros((nb, k), jnp.float32), x[:, :-k]], axis=1)
        k *= 2
    rank = jnp.sum(oht * x, axis=0, keepdims=True) - 1.0
    totals = x[:, n - 1 : n]
    t = totals
    k = 1
    while k < nb:
        t = t + jnp.concatenate(
            [jnp.zeros((k, 1), jnp.float32), t[:-k, :]], axis=0)
        k *= 2
    offs = t - totals
    posf = jnp.sum(oht * offs, axis=0, keepdims=True) + rank
    posi = posf.astype(jnp.int32)
    pos_ref[0] = posi + b * n
    irow = lax.broadcasted_iota(jnp.int32, (1, n), 1)
    jcol = lax.broadcasted_iota(jnp.int32, (128, 1), 0)
    qrow = lax.broadcasted_iota(jnp.int32, (n // 128, n), 0)
    lo_match = posi % 128 == jcol
    p_hi = jnp.where(lo_match, irow // 128, 0).astype(jnp.float32)
    p_lo = jnp.where(lo_match, irow % 128, 0).astype(jnp.float32)
    h_mat = (posi // 128 == qrow).astype(jnp.float32)
    ordt = (128.0 * jnp.dot(p_hi, h_mat.T, preferred_element_type=jnp.float32)
            + jnp.dot(p_lo, h_mat.T, preferred_element_type=jnp.float32))
    ordt_ref[0] = ordt.astype(jnp.int32)


def _sc_body(pos_hbm, feat_hbm, dist_hbm,
             featout_hbm, distout_hbm,
             idx_v, rows_v, drows_v, sem1, sem2):
    c = lax.axis_index("c")
    s = lax.axis_index("s")
    w = s * 2 + c
    base = w * 128
    pltpu.sync_copy(pos_hbm.at[pl.ds(base, 128)], idx_v)
    pltpu.sync_copy(feat_hbm.at[pl.ds(base, 128)], rows_v)
    pltpu.async_copy(rows_v, featout_hbm.at[idx_v], sem1).wait()
    pltpu.sync_copy(dist_hbm.at[pl.ds(base, 128)], drows_v)
    pltpu.async_copy(drows_v, distout_hbm.at[idx_v], sem2).wait()


def _mlp_body(a_ref, w1a_ref, w1b_ref, w2_ref, w3_ref,
              b1_ref, b2_ref, b3_ref, o_ref, *, dff: int, dd: int):
    A = a_ref[0][:, :dd]
    AT = A.T
    UT = jnp.dot(w1a_ref[...], AT, preferred_element_type=jnp.float32)
    VTb = jnp.dot(w1b_ref[...], AT,
                  preferred_element_type=jnp.float32) + b1_ref[...]
    EU = jnp.exp(UT)
    EV = jnp.exp(VTb)
    R = 32
    vtw = jnp.concatenate([VTb] * R, axis=1)
    evw = jnp.concatenate([EV] * R, axis=1)
    e_mat = (lax.broadcasted_iota(jnp.int32, (R, R * 128), 1) // 128
             == lax.broadcasted_iota(jnp.int32, (R, R * 128), 0)
             ).astype(jnp.float32)
    w2m, w3m = w2_ref[...], w3_ref[...]
    b2c, b3c = b2_ref[...], b3_ref[...]
    for i0 in range(0, 128, R):
        uw = jnp.dot(UT[:, i0 : i0 + R], e_mat,
                     preferred_element_type=jnp.float32)
        euw = jnp.dot(EU[:, i0 : i0 + R], e_mat,
                      preferred_element_type=jnp.float32)
        x1 = uw + vtw
        h = jnp.where(x1 > 0, x1, euw * evw - 1.0)
        h = _elu(jnp.dot(w2m, h, preferred_element_type=jnp.float32) + b2c)
        h = _elu(jnp.dot(w3m, h, preferred_element_type=jnp.float32) + b3c)
        for r in range(R):
            o_ref[0, 0, i0 + r, :, :] = h[:, r * 128 : (r + 1) * 128]


def kernel(x_dist, x_features, msk, codebook, W1, b1, W2, b2, W3, b3):
    batch, n, dd = x_dist.shape
    fd = x_features.shape[-1]
    dff = W1.shape[-1]
    nb = n // _BIN

    mul = jnp.matmul(x_dist, codebook[:, : nb // 2])
    cmul = jnp.concatenate([mul, -mul], axis=-1)
    bin_idx = jnp.argmax(cmul, axis=-1) + jnp.where(~msk, nb - 1, 0)

    pos3, ordt = pl.pallas_call(
        functools.partial(_pos_body, nb=nb),
        grid=(batch,),
        in_specs=[pl.BlockSpec((1, 1, n), lambda b: (b, 0, 0))],
        out_specs=[
            pl.BlockSpec((1, 1, n), lambda b: (b, 0, 0)),
            pl.BlockSpec((1, 128, n // 128), lambda b: (b, 0, 0)),
        ],
        out_shape=[
            jax.ShapeDtypeStruct((batch, 1, n), jnp.int32),
            jax.ShapeDtypeStruct((batch, 128, n // 128), jnp.int32),
        ],
    )(bin_idx.astype(jnp.int32).reshape(batch, 1, n))
    pos_flat = pos3.reshape(batch * n)
    order = ordt.transpose(0, 2, 1)

    mesh = plsc.VectorSubcoreMesh(core_axis_name="c", subcore_axis_name="s")
    sc_fn = pl.kernel(
        _sc_body,
        out_type=[
            jax.ShapeDtypeStruct((batch * n, fd), jnp.float32),
            jax.ShapeDtypeStruct((batch * n, 128), jnp.float32),
        ],
        mesh=mesh,
        scratch_types=[
            pltpu.VMEM((128,), jnp.int32),
            pltpu.VMEM((128, fd), jnp.float32),
            pltpu.VMEM((128, 128), jnp.float32),
            pltpu.SemaphoreType.DMA,
            pltpu.SemaphoreType.DMA,
        ],
    )
    xd_pad = jnp.pad(x_dist.reshape(batch * n, dd),
                     ((0, 0), (0, 128 - dd)))
    feat_b, dist_b = sc_fn(
        pos_flat, x_features.reshape(batch * n, fd), xd_pad)

    wspec = pl.BlockSpec((dff, dff), lambda g: (0, 0))
    bspec = pl.BlockSpec((dff, 1), lambda g: (0, 0))
    dm = pl.pallas_call(
        functools.partial(_mlp_body, dff=dff, dd=dd),
        grid=(batch * nb,),
        in_specs=[
            pl.BlockSpec((1, _BIN, 128), lambda g: (g, 0, 0)),
            wspec, wspec, wspec, wspec, bspec, bspec, bspec,
        ],
        out_specs=pl.BlockSpec((1, 1, _BIN, dff, _BIN),
                               lambda g: (g // nb, g % nb, 0, 0, 0)),
        out_shape=jax.ShapeDtypeStruct((batch, nb, _BIN, dff, _BIN),
                                       jnp.float32),
    )(dist_b.reshape(batch * nb, _BIN, 128),
      W1[:dd].T, W1[dd:].T, W2.T, W3.T,
      b1.reshape(dff, 1), b2.reshape(dff, 1), b3.reshape(dff, 1))
    dm = dm.transpose(0, 1, 2, 4, 3)

    bins_split = order.reshape(batch, nb, _BIN)
    xfb = feat_b.reshape(batch, nb, _BIN, fd)
    mskb = jnp.ones((batch, nb, _BIN, 1), x_dist.dtype)
    return (bins_split, xfb, dm, mskb)


# speedup vs baseline: 1.0153x; 1.0153x over previous
import functools

import jax
import jax.numpy as jnp
from jax import lax
from jax.experimental import pallas as pl
from jax.experimental.pallas import tpu as pltpu
from jax.experimental.pallas import tpu_sc as plsc

_BIN = 128


def _elu(x):
    return jnp.where(x > 0, x, jnp.exp(x) - 1.0)


def _pos_body(bi_ref, pos_ref, ordt_ref, *, nb: int):
    b = pl.program_id(0)
    n = bi_ref.shape[-1]
    bi = bi_ref[0]
    rows = lax.broadcasted_iota(jnp.int32, (nb, n), 0)
    oht = (bi == rows).astype(jnp.float32)
    x = oht
    k = 1
    while k < n:
        x = x + jnp.concatenate(
            [jnp.zeros((nb, k), jnp.float32), x[:, :-k]], axis=1)
        k *= 2
    rank = jnp.sum(oht * x, axis=0, keepdims=True) - 1.0
    totals = x[:, n - 1 : n]
    t = totals
    k = 1
    while k < nb:
        t = t + jnp.concatenate(
            [jnp.zeros((k, 1), jnp.float32), t[:-k, :]], axis=0)
        k *= 2
    offs = t - totals
    posf = jnp.sum(oht * offs, axis=0, keepdims=True) + rank
    posi = posf.astype(jnp.int32)
    pos_ref[0] = posi + b * n
    irow = lax.broadcasted_iota(jnp.int32, (1, n), 1)
    jcol = lax.broadcasted_iota(jnp.int32, (128, 1), 0)
    qrow = lax.broadcasted_iota(jnp.int32, (n // 128, n), 0)
    lo_match = posi % 128 == jcol
    p_hi = jnp.where(lo_match, irow // 128, 0).astype(jnp.float32)
    p_lo = jnp.where(lo_match, irow % 128, 0).astype(jnp.float32)
    h_mat = (posi // 128 == qrow).astype(jnp.float32)
    ordt = (128.0 * jnp.dot(p_hi, h_mat.T, preferred_element_type=jnp.float32)
            + jnp.dot(p_lo, h_mat.T, preferred_element_type=jnp.float32))
    ordt_ref[0] = ordt.astype(jnp.int32)


def _sc_body(pos_hbm, feat_hbm, dist_hbm,
             featout_hbm, distout_hbm,
             idx_v, rows_v, drows_v, sem1, sem2):
    c = lax.axis_index("c")
    s = lax.axis_index("s")
    w = s * 2 + c
    base = w * 128
    pltpu.sync_copy(pos_hbm.at[pl.ds(base, 128)], idx_v)
    pltpu.sync_copy(feat_hbm.at[pl.ds(base, 128)], rows_v)
    pltpu.async_copy(rows_v, featout_hbm.at[idx_v], sem1).wait()
    pltpu.sync_copy(dist_hbm.at[pl.ds(base, 128)], drows_v)
    pltpu.async_copy(drows_v, distout_hbm.at[idx_v], sem2).wait()


def _mlp_body(a_ref, w1a_ref, w1b_ref, w2_ref, w3_ref,
              b1_ref, b2_ref, b3_ref, o_ref, *, dff: int, dd: int):
    A = a_ref[0][:, :dd]
    AT = A.T
    UT = jnp.dot(w1a_ref[...], AT, preferred_element_type=jnp.float32)
    VTb = jnp.dot(w1b_ref[...], AT,
                  preferred_element_type=jnp.float32) + b1_ref[...]
    EU = jnp.exp(UT)
    EV = jnp.exp(VTb)
    R = 32
    w2m, w3m = w2_ref[...], w3_ref[...]
    b2c, b3c = b2_ref[...], b3_ref[...]
    for i0 in range(0, 128, R):
        x1 = jnp.concatenate(
            [UT[:, i0 + r : i0 + r + 1] + VTb for r in range(R)],
            axis=1)
        p1 = jnp.concatenate(
            [EU[:, i0 + r : i0 + r + 1] * EV for r in range(R)], axis=1)
        h = jnp.where(x1 > 0, x1, p1 - 1.0)
        h = _elu(jnp.dot(w2m, h, preferred_element_type=jnp.float32) + b2c)
        h = _elu(jnp.dot(w3m, h, preferred_element_type=jnp.float32) + b3c)
        for r in range(R):
            o_ref[0, 0, i0 + r, :, :] = h[:, r * 128 : (r + 1) * 128]


def kernel(x_dist, x_features, msk, codebook, W1, b1, W2, b2, W3, b3):
    batch, n, dd = x_dist.shape
    fd = x_features.shape[-1]
    dff = W1.shape[-1]
    nb = n // _BIN

    mul = jnp.matmul(x_dist, codebook[:, : nb // 2])
    cmul = jnp.concatenate([mul, -mul], axis=-1)
    bin_idx = jnp.argmax(cmul, axis=-1) + jnp.where(~msk, nb - 1, 0)

    pos3, ordt = pl.pallas_call(
        functools.partial(_pos_body, nb=nb),
        grid=(batch,),
        in_specs=[pl.BlockSpec((1, 1, n), lambda b: (b, 0, 0))],
        out_specs=[
            pl.BlockSpec((1, 1, n), lambda b: (b, 0, 0)),
            pl.BlockSpec((1, 128, n // 128), lambda b: (b, 0, 0)),
        ],
        out_shape=[
            jax.ShapeDtypeStruct((batch, 1, n), jnp.int32),
            jax.ShapeDtypeStruct((batch, 128, n // 128), jnp.int32),
        ],
    )(bin_idx.astype(jnp.int32).reshape(batch, 1, n))
    pos_flat = pos3.reshape(batch * n)
    order = ordt.transpose(0, 2, 1)

    mesh = plsc.VectorSubcoreMesh(core_axis_name="c", subcore_axis_name="s")
    sc_fn = pl.kernel(
        _sc_body,
        out_type=[
            jax.ShapeDtypeStruct((batch * n, fd), jnp.float32),
            jax.ShapeDtypeStruct((batch * n, 128), jnp.float32),
        ],
        mesh=mesh,
        scratch_types=[
            pltpu.VMEM((128,), jnp.int32),
            pltpu.VMEM((128, fd), jnp.float32),
            pltpu.VMEM((128, 128), jnp.float32),
            pltpu.SemaphoreType.DMA,
            pltpu.SemaphoreType.DMA,
        ],
    )
    xd_pad = jnp.pad(x_dist.reshape(batch * n, dd),
                     ((0, 0), (0, 128 - dd)))
    feat_b, dist_b = sc_fn(
        pos_flat, x_features.reshape(batch * n, fd), xd_pad)

    wspec = pl.BlockSpec((dff, dff), lambda g: (0, 0))
    bspec = pl.BlockSpec((dff, 1), lambda g: (0, 0))
    dm = pl.pallas_call(
        functools.partial(_mlp_body, dff=dff, dd=dd),
        grid=(batch * nb,),
        in_specs=[
            pl.BlockSpec((1, _BIN, 128), lambda g: (g, 0, 0)),
            wspec, wspec, wspec, wspec, bspec, bspec, bspec,
        ],
        out_specs=pl.BlockSpec((1, 1, _BIN, dff, _BIN),
                               lambda g: (g // nb, g % nb, 0, 0, 0)),
        out_shape=jax.ShapeDtypeStruct((batch, nb, _BIN, dff, _BIN),
                                       jnp.float32),
    )(dist_b.reshape(batch * nb, _BIN, 128),
      W1[:dd].T, W1[dd:].T, W2.T, W3.T,
      b1.reshape(dff, 1), b2.reshape(dff, 1), b3.reshape(dff, 1))
    dm = dm.transpose(0, 1, 2, 4, 3)

    bins_split = order.reshape(batch, nb, _BIN)
    xfb = feat_b.reshape(batch, nb, _BIN, fd)
    mskb = jnp.ones((batch, nb, _BIN, 1), x_dist.dtype)
    return (bins_split, xfb, dm, mskb)


# speedup vs baseline: 1.0538x; 1.0379x over previous
import functools

import jax
import jax.numpy as jnp
from jax import lax
from jax.experimental import pallas as pl
from jax.experimental.pallas import tpu as pltpu
from jax.experimental.pallas import tpu_sc as plsc

_BIN = 128


def _elu(x):
    return jnp.where(x > 0, x, jnp.exp(x) - 1.0)


def _pos_body(bi_ref, pos_ref, ordt_ref, *, nb: int):
    b = pl.program_id(0)
    n = bi_ref.shape[-1]
    bi = bi_ref[0]
    rows = lax.broadcasted_iota(jnp.int32, (nb, n), 0)
    oht = (bi == rows).astype(jnp.float32)
    x = oht
    k = 1
    while k < n:
        x = x + jnp.concatenate(
            [jnp.zeros((nb, k), jnp.float32), x[:, :-k]], axis=1)
        k *= 2
    rank = jnp.sum(oht * x, axis=0, keepdims=True) - 1.0
    totals = x[:, n - 1 : n]
    t = totals
    k = 1
    while k < nb:
        t = t + jnp.concatenate(
            [jnp.zeros((k, 1), jnp.float32), t[:-k, :]], axis=0)
        k *= 2
    offs = t - totals
    posf = jnp.sum(oht * offs, axis=0, keepdims=True) + rank
    posi = posf.astype(jnp.int32)
    pos_ref[0] = posi + b * n
    irow = lax.broadcasted_iota(jnp.int32, (1, n), 1)
    jcol = lax.broadcasted_iota(jnp.int32, (128, 1), 0)
    qrow = lax.broadcasted_iota(jnp.int32, (n // 128, n), 0)
    lo_match = posi % 128 == jcol
    p_hi = jnp.where(lo_match, irow // 128, 0).astype(jnp.float32)
    p_lo = jnp.where(lo_match, irow % 128, 0).astype(jnp.float32)
    h_mat = (posi // 128 == qrow).astype(jnp.float32)
    ordt = (128.0 * jnp.dot(p_hi, h_mat.T, preferred_element_type=jnp.float32)
            + jnp.dot(p_lo, h_mat.T, preferred_element_type=jnp.float32))
    ordt_ref[0] = ordt.astype(jnp.int32)


def _sc_body(pos_hbm, feat_hbm, dist_hbm,
             featout_hbm, distout_hbm,
             idx_v, rows_v, drows_v, sem1, sem2):
    c = lax.axis_index("c")
    s = lax.axis_index("s")
    w = s * 2 + c
    base = w * 128
    pltpu.sync_copy(pos_hbm.at[pl.ds(base, 128)], idx_v)
    pltpu.sync_copy(feat_hbm.at[pl.ds(base, 128)], rows_v)
    pltpu.async_copy(rows_v, featout_hbm.at[idx_v], sem1).wait()
    pltpu.sync_copy(dist_hbm.at[pl.ds(base, 128)], drows_v)
    pltpu.async_copy(drows_v, distout_hbm.at[idx_v], sem2).wait()


def _mlp_body(a_ref, w1a_ref, w1b_ref, w2_ref, w3_ref,
              b1_ref, b2_ref, b3_ref, o_ref, *, dff: int, dd: int):
    w2m, w3m = w2_ref[...], w3_ref[...]
    b2c, b3c = b2_ref[...], b3_ref[...]
    R = 32
    for pb in range(2):
        A = a_ref[0][pb * 128 : (pb + 1) * 128, :dd]
        AT = A.T
        UT = jnp.dot(w1a_ref[...], AT, preferred_element_type=jnp.float32)
        VTb = jnp.dot(w1b_ref[...], AT,
                      preferred_element_type=jnp.float32) + b1_ref[...]
        EU = jnp.exp(UT)
        EV = jnp.exp(VTb)
        for i0 in range(0, 128, R):
            x1 = jnp.concatenate(
                [UT[:, i0 + r : i0 + r + 1] + VTb for r in range(R)],
                axis=1)
            p1 = jnp.concatenate(
                [EU[:, i0 + r : i0 + r + 1] * EV for r in range(R)], axis=1)
            h = jnp.where(x1 > 0, x1, p1 - 1.0)
            h = _elu(jnp.dot(w2m, h, preferred_element_type=jnp.float32)
                     + b2c)
            h = _elu(jnp.dot(w3m, h, preferred_element_type=jnp.float32)
                     + b3c)
            for r in range(R):
                o_ref[0, pb, i0 + r, :, :] = h[:, r * 128 : (r + 1) * 128]


def kernel(x_dist, x_features, msk, codebook, W1, b1, W2, b2, W3, b3):
    batch, n, dd = x_dist.shape
    fd = x_features.shape[-1]
    dff = W1.shape[-1]
    nb = n // _BIN

    mul = jnp.matmul(x_dist, codebook[:, : nb // 2])
    cmul = jnp.concatenate([mul, -mul], axis=-1)
    bin_idx = jnp.argmax(cmul, axis=-1) + jnp.where(~msk, nb - 1, 0)

    pos3, ordt = pl.pallas_call(
        functools.partial(_pos_body, nb=nb),
        grid=(batch,),
        in_specs=[pl.BlockSpec((1, 1, n), lambda b: (b, 0, 0))],
        out_specs=[
            pl.BlockSpec((1, 1, n), lambda b: (b, 0, 0)),
            pl.BlockSpec((1, 128, n // 128), lambda b: (b, 0, 0)),
        ],
        out_shape=[
            jax.ShapeDtypeStruct((batch, 1, n), jnp.int32),
            jax.ShapeDtypeStruct((batch, 128, n // 128), jnp.int32),
        ],
    )(bin_idx.astype(jnp.int32).reshape(batch, 1, n))
    pos_flat = pos3.reshape(batch * n)
    order = ordt.transpose(0, 2, 1)

    mesh = plsc.VectorSubcoreMesh(core_axis_name="c", subcore_axis_name="s")
    sc_fn = pl.kernel(
        _sc_body,
        out_type=[
            jax.ShapeDtypeStruct((batch * n, fd), jnp.float32),
            jax.ShapeDtypeStruct((batch * n, 128), jnp.float32),
        ],
        mesh=mesh,
        scratch_types=[
            pltpu.VMEM((128,), jnp.int32),
            pltpu.VMEM((128, fd), jnp.float32),
            pltpu.VMEM((128, 128), jnp.float32),
            pltpu.SemaphoreType.DMA,
            pltpu.SemaphoreType.DMA,
        ],
    )
    xd_pad = jnp.pad(x_dist.reshape(batch * n, dd),
                     ((0, 0), (0, 128 - dd)))
    feat_b, dist_b = sc_fn(
        pos_flat, x_features.reshape(batch * n, fd), xd_pad)

    wspec = pl.BlockSpec((dff, dff), lambda g: (0, 0))
    bspec = pl.BlockSpec((dff, 1), lambda g: (0, 0))
    dm = pl.pallas_call(
        functools.partial(_mlp_body, dff=dff, dd=dd),
        grid=(batch * nb // 2,),
        in_specs=[
            pl.BlockSpec((1, 2 * _BIN, 128), lambda g: (g, 0, 0)),
            wspec, wspec, wspec, wspec, bspec, bspec, bspec,
        ],
        out_specs=pl.BlockSpec((1, 2, _BIN, dff, _BIN),
                               lambda g: (g // (nb // 2), g % (nb // 2),
                                          0, 0, 0)),
        out_shape=jax.ShapeDtypeStruct((batch, nb, _BIN, dff, _BIN),
                                       jnp.float32),
    )(dist_b.reshape(batch * nb // 2, 2 * _BIN, 128),
      W1[:dd].T, W1[dd:].T, W2.T, W3.T,
      b1.reshape(dff, 1), b2.reshape(dff, 1), b3.reshape(dff, 1))
    dm = dm.transpose(0, 1, 2, 4, 3)

    bins_split = order.reshape(batch, nb, _BIN)
    xfb = feat_b.reshape(batch, nb, _BIN, fd)
    mskb = jnp.ones((batch, nb, _BIN, 1), x_dist.dtype)
    return (bins_split, xfb, dm, mskb)


# speedup vs baseline: 1.0667x; 1.0122x over previous
import functools

import jax
import jax.numpy as jnp
from jax import lax
from jax.experimental import pallas as pl
from jax.experimental.pallas import tpu as pltpu
from jax.experimental.pallas import tpu_sc as plsc

_BIN = 128


def _elu(x):
    return jnp.where(x > 0, x, jnp.exp(x) - 1.0)


def _pos_body(bi_ref, pos_ref, ordt_ref, *, nb: int):
    b = pl.program_id(0)
    n = bi_ref.shape[-1]
    bi = bi_ref[0]
    rows = lax.broadcasted_iota(jnp.int32, (nb, n), 0)
    oht = (bi == rows).astype(jnp.float32)
    x = oht
    k = 1
    while k < n:
        x = x + jnp.concatenate(
            [jnp.zeros((nb, k), jnp.float32), x[:, :-k]], axis=1)
        k *= 2
    rank = jnp.sum(oht * x, axis=0, keepdims=True) - 1.0
    totals = x[:, n - 1 : n]
    t = totals
    k = 1
    while k < nb:
        t = t + jnp.concatenate(
            [jnp.zeros((k, 1), jnp.float32), t[:-k, :]], axis=0)
        k *= 2
    offs = t - totals
    posf = jnp.sum(oht * offs, axis=0, keepdims=True) + rank
    posi = posf.astype(jnp.int32)
    pos_ref[0] = posi + b * n
    irow = lax.broadcasted_iota(jnp.int32, (1, n), 1)
    jcol = lax.broadcasted_iota(jnp.int32, (128, 1), 0)
    qrow = lax.broadcasted_iota(jnp.int32, (n // 128, n), 0)
    lo_match = posi % 128 == jcol
    p_hi = jnp.where(lo_match, irow // 128, 0).astype(jnp.float32)
    p_lo = jnp.where(lo_match, irow % 128, 0).astype(jnp.float32)
    h_mat = (posi // 128 == qrow).astype(jnp.float32)
    ordt = (128.0 * jnp.dot(p_hi, h_mat.T, preferred_element_type=jnp.float32)
            + jnp.dot(p_lo, h_mat.T, preferred_element_type=jnp.float32))
    ordt_ref[0] = ordt.astype(jnp.int32)


def _sc_body(pos_hbm, feat_hbm, dist_hbm,
             featout_hbm, distout_hbm,
             idx_v, rows_v, drows_v, sem1, sem2):
    c = lax.axis_index("c")
    s = lax.axis_index("s")
    w = s * 2 + c
    base = w * 128
    pltpu.sync_copy(pos_hbm.at[pl.ds(base, 128)], idx_v)
    pltpu.sync_copy(feat_hbm.at[pl.ds(base, 128)], rows_v)
    pltpu.async_copy(rows_v, featout_hbm.at[idx_v], sem1).wait()
    pltpu.sync_copy(dist_hbm.at[pl.ds(base, 128)], drows_v)
    pltpu.async_copy(drows_v, distout_hbm.at[idx_v], sem2).wait()


def _mlp_body(a_ref, w1a_ref, w1b_ref, w2_ref, w3_ref,
              b1_ref, b2_ref, b3_ref, o_ref, *, dff: int, dd: int):
    w2m, w3m = w2_ref[...], w3_ref[...]
    b2c, b3c = b2_ref[...], b3_ref[...]
    R = 32
    for pb in range(4):
        A = a_ref[0][pb * 128 : (pb + 1) * 128, :dd]
        AT = A.T
        UT = jnp.dot(w1a_ref[...], AT, preferred_element_type=jnp.float32)
        VTb = jnp.dot(w1b_ref[...], AT,
                      preferred_element_type=jnp.float32) + b1_ref[...]
        EU = jnp.exp(UT)
        EV = jnp.exp(VTb)
        for i0 in range(0, 128, R):
            x1 = jnp.concatenate(
                [UT[:, i0 + r : i0 + r + 1] + VTb for r in range(R)],
                axis=1)
            p1 = jnp.concatenate(
                [EU[:, i0 + r : i0 + r + 1] * EV for r in range(R)], axis=1)
            h = jnp.where(x1 > 0, x1, p1 - 1.0)
            h = _elu(jnp.dot(w2m, h, preferred_element_type=jnp.float32)
                     + b2c)
            h = _elu(jnp.dot(w3m, h, preferred_element_type=jnp.float32)
                     + b3c)
            for r in range(R):
                o_ref[0, pb, i0 + r, :, :] = h[:, r * 128 : (r + 1) * 128]


def kernel(x_dist, x_features, msk, codebook, W1, b1, W2, b2, W3, b3):
    batch, n, dd = x_dist.shape
    fd = x_features.shape[-1]
    dff = W1.shape[-1]
    nb = n // _BIN

    mul = jnp.matmul(x_dist, codebook[:, : nb // 2])
    cmul = jnp.concatenate([mul, -mul], axis=-1)
    bin_idx = jnp.argmax(cmul, axis=-1) + jnp.where(~msk, nb - 1, 0)

    pos3, ordt = pl.pallas_call(
        functools.partial(_pos_body, nb=nb),
        grid=(batch,),
        in_specs=[pl.BlockSpec((1, 1, n), lambda b: (b, 0, 0))],
        out_specs=[
            pl.BlockSpec((1, 1, n), lambda b: (b, 0, 0)),
            pl.BlockSpec((1, 128, n // 128), lambda b: (b, 0, 0)),
        ],
        out_shape=[
            jax.ShapeDtypeStruct((batch, 1, n), jnp.int32),
            jax.ShapeDtypeStruct((batch, 128, n // 128), jnp.int32),
        ],
    )(bin_idx.astype(jnp.int32).reshape(batch, 1, n))
    pos_flat = pos3.reshape(batch * n)
    order = ordt.transpose(0, 2, 1)

    mesh = plsc.VectorSubcoreMesh(core_axis_name="c", subcore_axis_name="s")
    sc_fn = pl.kernel(
        _sc_body,
        out_type=[
            jax.ShapeDtypeStruct((batch * n, fd), jnp.float32),
            jax.ShapeDtypeStruct((batch * n, 128), jnp.float32),
        ],
        mesh=mesh,
        scratch_types=[
            pltpu.VMEM((128,), jnp.int32),
            pltpu.VMEM((128, fd), jnp.float32),
            pltpu.VMEM((128, 128), jnp.float32),
            pltpu.SemaphoreType.DMA,
            pltpu.SemaphoreType.DMA,
        ],
    )
    xd_pad = jnp.pad(x_dist.reshape(batch * n, dd),
                     ((0, 0), (0, 128 - dd)))
    feat_b, dist_b = sc_fn(
        pos_flat, x_features.reshape(batch * n, fd), xd_pad)

    wspec = pl.BlockSpec((dff, dff), lambda g: (0, 0))
    bspec = pl.BlockSpec((dff, 1), lambda g: (0, 0))
    dm = pl.pallas_call(
        functools.partial(_mlp_body, dff=dff, dd=dd),
        grid=(batch * nb // 4,),
        in_specs=[
            pl.BlockSpec((1, 4 * _BIN, 128), lambda g: (g, 0, 0)),
            wspec, wspec, wspec, wspec, bspec, bspec, bspec,
        ],
        out_specs=pl.BlockSpec((1, 4, _BIN, dff, _BIN),
                               lambda g: (g // (nb // 4), g % (nb // 4),
                                          0, 0, 0)),
        out_shape=jax.ShapeDtypeStruct((batch, nb, _BIN, dff, _BIN),
                                       jnp.float32),
    )(dist_b.reshape(batch * nb // 4, 4 * _BIN, 128),
      W1[:dd].T, W1[dd:].T, W2.T, W3.T,
      b1.reshape(dff, 1), b2.reshape(dff, 1), b3.reshape(dff, 1))
    dm = dm.transpose(0, 1, 2, 4, 3)

    bins_split = order.reshape(batch, nb, _BIN)
    xfb = feat_b.reshape(batch, nb, _BIN, fd)
    mskb = jnp.ones((batch, nb, _BIN, 1), x_dist.dtype)
    return (bins_split, xfb, dm, mskb)
